# baseline (speedup 1.0000x reference)
import functools

import jax
import jax.numpy as jnp
from jax import lax
from jax.experimental import pallas as pl
from jax.experimental.pallas import tpu as pltpu
from jax.experimental.pallas import tpu_sc as plsc

_B = 16384
_D = 16
_NNEG = 20
_NW = 32
_RPW = _B // _NW
_CH = 128
_NCH = _RPW // _CH
_NG = _CH // _D


def _bpr_body(users_h, pos_h, negt_h, uf_h, if_h,
              acc_h, pqu_h, ppi_h, npi_h,
              uidx_v, pidx_v, nidx_v, u_v, p_v, neg_v,
              pos_v, pqu_v, ppi_v, npi_v, acc_v,
              sem_g, sem_n0, sem_n1):
  cid = lax.axis_index("c")
  sid = lax.axis_index("s")
  wid = sid * 2 + cid
  base = wid * _RPW

  iota = lax.iota(jnp.int32, _D)

  pltpu.sync_copy(users_h.at[pl.ds(base, _RPW)], uidx_v)
  pltpu.sync_copy(pos_h.at[pl.ds(base, _RPW)], pidx_v)
  for n in range(_NNEG):
    pltpu.sync_copy(negt_h.at[n, pl.ds(base, _RPW)], nidx_v.at[n])

  gdescs = []
  for t in range(_NCH):
    sl = pl.ds(t * _CH, _CH)
    gdescs.append(pltpu.async_copy(uf_h.at[uidx_v.at[sl]], u_v.at[sl], sem_g))
    gdescs.append(pltpu.async_copy(if_h.at[pidx_v.at[sl]], p_v.at[sl], sem_g))

  sems = (sem_n0, sem_n1)

  def fire_chunk(c):
    slot = c % 2
    descs = []
    for n in range(_NNEG):
      descs.append(pltpu.async_copy(
          if_h.at[nidx_v.at[n, pl.ds(c * _CH, _CH)]],
          neg_v.at[slot, n], sems[slot]))
    return descs

  ndescs = fire_chunk(0)

  for d in gdescs:
    d.wait()

  @plsc.parallel_loop(0, _RPW // _D, unroll=2)
  def _pos_loop(j):
    rows = iota + j * _D
    s_up = jnp.zeros((_D,), jnp.float32)
    s_uu = jnp.zeros((_D,), jnp.float32)
    s_pp = jnp.zeros((_D,), jnp.float32)
    for d in range(_D):
      cols = (iota + d) & (_D - 1)
      uv = plsc.load_gather(u_v, [rows, cols])
      pv = plsc.load_gather(p_v, [rows, cols])
      s_up = s_up + uv * pv
      s_uu = s_uu + uv * uv
      s_pp = s_pp + pv * pv
    r0 = j * _D
    pos_v[pl.ds(r0, _D)] = s_up
    pqu_v[pl.ds(r0, _D)] = s_uu
    ppi_v[pl.ds(r0, _D)] = s_pp

  for c in range(_NCH):
    slot = c % 2
    cur_descs = ndescs
    if c + 1 < _NCH:
      ndescs = fire_chunk(c + 1)
    for d in cur_descs:
      d.wait()
    nslot_v = neg_v.at[slot]

    @plsc.parallel_loop(0, _NNEG * _NG, unroll=2)
    def _neg_loop(g):
      n = g >> 3
      j = g & (_NG - 1)
      lrows = iota + j * _D
      grows = lrows + c * _CH
      nvec = jnp.full((_D,), n, jnp.int32)
      s0 = jnp.zeros((_D,), jnp.float32)
      s1 = jnp.zeros((_D,), jnp.float32)
      for d in range(0, _D, 2):
        cols0 = (iota + d) & (_D - 1)
        cols1 = (iota + d + 1) & (_D - 1)
        s0 = s0 + (plsc.load_gather(u_v, [grows, cols0])
                   * plsc.load_gather(nslot_v, [nvec, lrows, cols0]))
        s1 = s1 + (plsc.load_gather(u_v, [grows, cols1])
                   * plsc.load_gather(nslot_v, [nvec, lrows, cols1]))
      pos = pos_v[pl.ds(c * _CH + j * _D, _D)]
      acc_v[pl.ds(n * _RPW + c * _CH + j * _D, _D)] = pos - (s0 + s1)

    @plsc.parallel_loop(0, _NG, unroll=2)
    def _npi_loop(j):
      lrows = iota + j * _D
      s_nn = jnp.zeros((_D,), jnp.float32)
      for d in range(_D):
        cols = (iota + d) & (_D - 1)
        nv = plsc.load_gather(nslot_v, [jnp.full((_D,), _NNEG - 1, jnp.int32),
                                        lrows, cols])
        s_nn = s_nn + nv * nv
      npi_v[pl.ds(c * _CH + j * _D, _D)] = s_nn

  for n in range(_NNEG):
    pltpu.sync_copy(acc_v.at[pl.ds(n * _RPW, _RPW)],
                    acc_h.at[n, pl.ds(base, _RPW)])
  pltpu.sync_copy(pqu_v, pqu_h.at[pl.ds(base, _RPW)])
  pltpu.sync_copy(ppi_v, ppi_h.at[pl.ds(base, _RPW)])
  pltpu.sync_copy(npi_v, npi_h.at[pl.ds(base, _RPW)])


_bpr = functools.partial(
    pl.kernel,
    out_type=[
        jax.ShapeDtypeStruct((_NNEG, _B), jnp.float32),
        jax.ShapeDtypeStruct((_B,), jnp.float32),
        jax.ShapeDtypeStruct((_B,), jnp.float32),
        jax.ShapeDtypeStruct((_B,), jnp.float32),
    ],
    mesh=plsc.VectorSubcoreMesh(core_axis_name="c", subcore_axis_name="s"),
    compiler_params=pltpu.CompilerParams(needs_layout_passes=False,
                                         use_tc_tiling_on_sc=False),
    scratch_types=[
        pltpu.VMEM((_RPW,), jnp.int32),
        pltpu.VMEM((_RPW,), jnp.int32),
        pltpu.VMEM((_NNEG, _RPW), jnp.int32),
        pltpu.VMEM((_RPW, _D), jnp.float32),
        pltpu.VMEM((_RPW, _D), jnp.float32),
        pltpu.VMEM((2, _NNEG, _CH, _D), jnp.float32),
        pltpu.VMEM((_RPW,), jnp.float32),
        pltpu.VMEM((_RPW,), jnp.float32),
        pltpu.VMEM((_RPW,), jnp.float32),
        pltpu.VMEM((_RPW,), jnp.float32),
        pltpu.VMEM((_NNEG * _RPW,), jnp.float32),
        pltpu.SemaphoreType.DMA,
        pltpu.SemaphoreType.DMA,
        pltpu.SemaphoreType.DMA,
    ],
)(_bpr_body)


@jax.jit
def kernel(users, pos_items, neg_items, user_factors, item_factors):
  users = users.astype(jnp.int32)
  pos_items = pos_items.astype(jnp.int32)
  neg_t = neg_items.astype(jnp.int32).T
  acc2d, pqu, ppi, npi = _bpr(users, pos_items, neg_t,
                              user_factors, item_factors)
  return (acc2d.reshape(-1), (pqu, ppi, pqu, npi))


# speedup vs baseline: 1.0083x; 1.0083x over previous
import functools

import jax
import jax.numpy as jnp
from jax import lax
from jax.experimental import pallas as pl
from jax.experimental.pallas import tpu as pltpu
from jax.experimental.pallas import tpu_sc as plsc

_B = 16384
_D = 16
_NNEG = 20
_NW = 32
_RPW = _B // _NW
_NG = _RPW // _D
_RING = 8


def _bpr_body(users_h, pos_h, neg_h, uf_h, if_h,
              acc_h, pqu_h, ppi_h, npi_h,
              uidx_v, pidx_v, nidx2_v, nidx_v, u_v, p_v, neg_v,
              pos_v, pqu_v, ppi_v, npi_v, acc_v,
              sem_u, sem_p, *sem_n):
  cid = lax.axis_index("c")
  sid = lax.axis_index("s")
  wid = sid * 2 + cid
  base = wid * _RPW

  iota = lax.iota(jnp.int32, _D)

  pltpu.sync_copy(users_h.at[pl.ds(base, _RPW)], uidx_v)
  pltpu.sync_copy(pos_h.at[pl.ds(base, _RPW)], pidx_v)
  pltpu.sync_copy(neg_h.at[pl.ds(base, _RPW), :], nidx2_v)

  du = pltpu.async_copy(uf_h.at[uidx_v], u_v, sem_u)
  dp = pltpu.async_copy(if_h.at[pidx_v], p_v, sem_p)

  @plsc.parallel_loop(0, _NNEG * _NG, unroll=4)
  def _tr_loop(g):
    n = g >> 5
    j = g & (_NG - 1)
    col = jnp.zeros((_D,), jnp.int32) + n
    v = plsc.load_gather(nidx2_v, [iota + j * _D, col])
    nidx_v[pl.ds(n * _RPW + j * _D, _D)] = v

  def fire(n):
    return pltpu.async_copy(
        if_h.at[nidx_v.at[pl.ds(n * _RPW, _RPW)]],
        neg_v.at[n % _RING], sem_n[n % _RING])

  ndescs = [fire(n) for n in range(_RING)]

  du.wait()
  dp.wait()

  @plsc.parallel_loop(0, _NG, unroll=2)
  def _pos_loop(j):
    rows = iota + j * _D
    s_up = jnp.zeros((_D,), jnp.float32)
    s_uu = jnp.zeros((_D,), jnp.float32)
    s_pp = jnp.zeros((_D,), jnp.float32)
    for d in range(_D):
      cols = (iota + d) & (_D - 1)
      uv = plsc.load_gather(u_v, [rows, cols])
      pv = plsc.load_gather(p_v, [rows, cols])
      s_up = s_up + uv * pv
      s_uu = s_uu + uv * uv
      s_pp = s_pp + pv * pv
    r0 = j * _D
    pos_v[pl.ds(r0, _D)] = s_up
    pqu_v[pl.ds(r0, _D)] = s_uu
    ppi_v[pl.ds(r0, _D)] = s_pp

  for n in range(_NNEG):
    slot = n % _RING
    ndescs[n].wait()
    nslot_v = neg_v.at[slot]

    @plsc.parallel_loop(0, _NG, unroll=2)
    def _neg_loop(j):
      rows = iota + j * _D
      s0 = jnp.zeros((_D,), jnp.float32)
      s1 = jnp.zeros((_D,), jnp.float32)
      for d in range(0, _D, 2):
        cols0 = (iota + d) & (_D - 1)
        cols1 = (iota + d + 1) & (_D - 1)
        s0 = s0 + (plsc.load_gather(u_v, [rows, cols0])
                   * plsc.load_gather(nslot_v, [rows, cols0]))
        s1 = s1 + (plsc.load_gather(u_v, [rows, cols1])
                   * plsc.load_gather(nslot_v, [rows, cols1]))
      pos = pos_v[pl.ds(j * _D, _D)]
      acc_v[pl.ds(n * _RPW + j * _D, _D)] = pos - (s0 + s1)

    if n == _NNEG - 1:
      @plsc.parallel_loop(0, _NG, unroll=2)
      def _npi_loop(j):
        rows = iota + j * _D
        s_nn = jnp.zeros((_D,), jnp.float32)
        for d in range(_D):
          cols = (iota + d) & (_D - 1)
          nv = plsc.load_gather(nslot_v, [rows, cols])
          s_nn = s_nn + nv * nv
        npi_v[pl.ds(j * _D, _D)] = s_nn

    if n + _RING < _NNEG:
      ndescs.append(fire(n + _RING))

  for n in range(_NNEG):
    pltpu.sync_copy(acc_v.at[pl.ds(n * _RPW, _RPW)],
                    acc_h.at[pl.ds(n * _B + base, _RPW)])
  pltpu.sync_copy(pqu_v, pqu_h.at[pl.ds(base, _RPW)])
  pltpu.sync_copy(ppi_v, ppi_h.at[pl.ds(base, _RPW)])
  pltpu.sync_copy(npi_v, npi_h.at[pl.ds(base, _RPW)])


_bpr = functools.partial(
    pl.kernel,
    out_type=[
        jax.ShapeDtypeStruct((_NNEG * _B,), jnp.float32),
        jax.ShapeDtypeStruct((_B,), jnp.float32),
        jax.ShapeDtypeStruct((_B,), jnp.float32),
        jax.ShapeDtypeStruct((_B,), jnp.float32),
    ],
    mesh=plsc.VectorSubcoreMesh(core_axis_name="c", subcore_axis_name="s"),
    compiler_params=pltpu.CompilerParams(needs_layout_passes=False,
                                         use_tc_tiling_on_sc=False),
    scratch_types=[
        pltpu.VMEM((_RPW,), jnp.int32),
        pltpu.VMEM((_RPW,), jnp.int32),
        pltpu.VMEM((_RPW, _NNEG), jnp.int32),
        pltpu.VMEM((_NNEG * _RPW,), jnp.int32),
        pltpu.VMEM((_RPW, _D), jnp.float32),
        pltpu.VMEM((_RPW, _D), jnp.float32),
        pltpu.VMEM((_RING, _RPW, _D), jnp.float32),
        pltpu.VMEM((_RPW,), jnp.float32),
        pltpu.VMEM((_RPW,), jnp.float32),
        pltpu.VMEM((_RPW,), jnp.float32),
        pltpu.VMEM((_RPW,), jnp.float32),
        pltpu.VMEM((_NNEG * _RPW,), jnp.float32),
        pltpu.SemaphoreType.DMA,
        pltpu.SemaphoreType.DMA,
    ] + [pltpu.SemaphoreType.DMA] * _RING,
)(_bpr_body)


@jax.jit
def kernel(users, pos_items, neg_items, user_factors, item_factors):
  users = users.astype(jnp.int32)
  pos_items = pos_items.astype(jnp.int32)
  neg_items = neg_items.astype(jnp.int32)
  acc, pqu, ppi, npi = _bpr(users, pos_items, neg_items,
                            user_factors, item_factors)
  return (acc, (pqu, ppi, pqu, npi))


# speedup vs baseline: 4.0705x; 4.0369x over previous
import functools

import jax
import jax.numpy as jnp
from jax import lax
from jax.experimental import pallas as pl
from jax.experimental.pallas import tpu as pltpu
from jax.experimental.pallas import tpu_sc as plsc

_B = 16384
_D = 16
_NNEG = 20
_NW = 32
_RPW = _B // _NW
_NG = _RPW // _D
_RING = 8

_NROW = 1000000
_NBLK = 7813
_NROWP = _NBLK * 128
_CBLK = 5
_NCHK = 49
_CW = _CBLK * 128
_CWORDS = _CW * _D


def _conv_body(uft_h, ift_h, ufl_h, ifl_h, tb0_v, tb1_v, ob0_v, ob1_v,
               sem_i0, sem_i1, sem_o0, sem_o1):
  tb_refs = (tb0_v, tb1_v)
  ob_refs = (ob0_v, ob1_v)
  cid = lax.axis_index("c")
  sid = lax.axis_index("s")
  wid = sid * 2 + cid
  start = jnp.where(wid < 31, wid * 245, _NBLK - 245)
  iota = lax.iota(jnp.int32, _D)
  sem_i = (sem_i0, sem_i1)
  sem_o = (sem_o0, sem_o1)

  for src_h, dst_h in ((uft_h, ufl_h), (ift_h, ifl_h)):

    def in_copy(k, s):
      return pltpu.make_async_copy(
          src_h.at[:, pl.ds((start + k * _CBLK) * 128, _CW)],
          tb_refs[s], sem_i[s])

    def out_copy(k, s):
      return pltpu.make_async_copy(
          ob_refs[s],
          dst_h.at[pl.ds((start + k * _CBLK) * (128 * _D), _CWORDS)],
          sem_o[s])

    in_copy(0, 0).start()

    def step(k0, _):
      for s in range(2):
        k = k0 * 2 + s

        @pl.when(k < _NCHK)
        def _():
          @pl.when(k + 1 < _NCHK)
          def _():
            in_copy(k + 1, 1 - s).start()

          in_copy(k, s).wait()

          @pl.when(k >= 2)
          def _():
            out_copy(k - 2, s).wait()

          tbc = tb_refs[s]
          ob = ob_refs[s]

          @plsc.parallel_loop(0, _CBLK * 8 * 16, unroll=4)
          def _rearrange(q):
            rows = (iota + q) & (_D - 1)
            cols = iota + ((q >> 4) << 4)
            val = plsc.load_gather(tbc, [rows, cols])
            dst = ((q >> 4) << 8) + (iota << 4) + rows
            plsc.store_scatter(ob, [dst], val)

          out_copy(k, s).start()
      return 0

    lax.fori_loop(0, (_NCHK + 1) // 2, step, 0)
    out_copy(_NCHK - 2, 1).wait()
    out_copy(_NCHK - 1, 0).wait()


_conv = functools.partial(
    pl.kernel,
    out_type=[
        jax.ShapeDtypeStruct((_NROWP * _D,), jnp.float32),
        jax.ShapeDtypeStruct((_NROWP * _D,), jnp.float32),
    ],
    mesh=plsc.VectorSubcoreMesh(core_axis_name="c", subcore_axis_name="s"),
    compiler_params=pltpu.CompilerParams(needs_layout_passes=False,
                                         use_tc_tiling_on_sc=True),
    scratch_types=[
        pltpu.VMEM((_D, _CW), jnp.float32),
        pltpu.VMEM((_D, _CW), jnp.float32),
        pltpu.VMEM((_CWORDS,), jnp.float32),
        pltpu.VMEM((_CWORDS,), jnp.float32),
        pltpu.SemaphoreType.DMA,
        pltpu.SemaphoreType.DMA,
        pltpu.SemaphoreType.DMA,
        pltpu.SemaphoreType.DMA,
    ],
)(_conv_body)


def _bpr_body(users_h, pos_h, neg_h, uf_h, if_h,
              acc_h, pqu_h, ppi_h, npi_h,
              uidx_v, pidx_v, nidx2_v, nidx_v, u_v, p_v, neg_v,
              pos_v, pqu_v, ppi_v, npi_v, acc_v,
              sem_u, sem_p, *sem_n):
  cid = lax.axis_index("c")
  sid = lax.axis_index("s")
  wid = sid * 2 + cid
  base = wid * _RPW

  iota = lax.iota(jnp.int32, _D)

  pltpu.sync_copy(users_h.at[pl.ds(base, _RPW)], uidx_v)
  pltpu.sync_copy(pos_h.at[pl.ds(base, _RPW)], pidx_v)
  pltpu.sync_copy(neg_h.at[pl.ds(base * _NNEG, _RPW * _NNEG)], nidx2_v)

  du = pltpu.async_copy(uf_h.at[uidx_v], u_v, sem_u)
  dp = pltpu.async_copy(if_h.at[pidx_v], p_v, sem_p)

  @plsc.parallel_loop(0, _NNEG * _NG, unroll=4)
  def _tr_loop(g):
    n = g >> 5
    j = g & (_NG - 1)
    v = plsc.load_gather(nidx2_v, [(iota + j * _D) * _NNEG + n])
    nidx_v[pl.ds(n * _RPW + j * _D, _D)] = v

  def fire(n):
    return pltpu.async_copy(
        if_h.at[nidx_v.at[pl.ds(n * _RPW, _RPW)]],
        neg_v.at[n % _RING], sem_n[n % _RING])

  ndescs = [fire(n) for n in range(_RING)]

  du.wait()
  dp.wait()

  @plsc.parallel_loop(0, _NG, unroll=2)
  def _pos_loop(j):
    rows = iota + j * _D
    s_up = jnp.zeros((_D,), jnp.float32)
    s_uu = jnp.zeros((_D,), jnp.float32)
    s_pp = jnp.zeros((_D,), jnp.float32)
    for d in range(_D):
      cols = (iota + d) & (_D - 1)
      uv = plsc.load_gather(u_v, [rows, cols])
      pv = plsc.load_gather(p_v, [rows, cols])
      s_up = s_up + uv * pv
      s_uu = s_uu + uv * uv
      s_pp = s_pp + pv * pv
    r0 = j * _D
    pos_v[pl.ds(r0, _D)] = s_up
    pqu_v[pl.ds(r0, _D)] = s_uu
    ppi_v[pl.ds(r0, _D)] = s_pp

  for n in range(_NNEG):
    slot = n % _RING
    ndescs[n].wait()
    nslot_v = neg_v.at[slot]

    @plsc.parallel_loop(0, _NG, unroll=2)
    def _neg_loop(j):
      rows = iota + j * _D
      s0 = jnp.zeros((_D,), jnp.float32)
      s1 = jnp.zeros((_D,), jnp.float32)
      for d in range(0, _D, 2):
        cols0 = (iota + d) & (_D - 1)
        cols1 = (iota + d + 1) & (_D - 1)
        s0 = s0 + (plsc.load_gather(u_v, [rows, cols0])
                   * plsc.load_gather(nslot_v, [rows, cols0]))
        s1 = s1 + (plsc.load_gather(u_v, [rows, cols1])
                   * plsc.load_gather(nslot_v, [rows, cols1]))
      pos = pos_v[pl.ds(j * _D, _D)]
      acc_v[pl.ds(n * _RPW + j * _D, _D)] = pos - (s0 + s1)

    if n == _NNEG - 1:
      @plsc.parallel_loop(0, _NG, unroll=2)
      def _npi_loop(j):
        rows = iota + j * _D
        s_nn = jnp.zeros((_D,), jnp.float32)
        for d in range(_D):
          cols = (iota + d) & (_D - 1)
          nv = plsc.load_gather(nslot_v, [rows, cols])
          s_nn = s_nn + nv * nv
        npi_v[pl.ds(j * _D, _D)] = s_nn

    if n + _RING < _NNEG:
      ndescs.append(fire(n + _RING))

  for n in range(_NNEG):
    pltpu.sync_copy(acc_v.at[pl.ds(n * _RPW, _RPW)],
                    acc_h.at[pl.ds(n * _B + base, _RPW)])
  pltpu.sync_copy(pqu_v, pqu_h.at[pl.ds(base, _RPW)])
  pltpu.sync_copy(ppi_v, ppi_h.at[pl.ds(base, _RPW)])
  pltpu.sync_copy(npi_v, npi_h.at[pl.ds(base, _RPW)])


_bpr = functools.partial(
    pl.kernel,
    out_type=[
        jax.ShapeDtypeStruct((_NNEG * _B,), jnp.float32),
        jax.ShapeDtypeStruct((_B,), jnp.float32),
        jax.ShapeDtypeStruct((_B,), jnp.float32),
        jax.ShapeDtypeStruct((_B,), jnp.float32),
    ],
    mesh=plsc.VectorSubcoreMesh(core_axis_name="c", subcore_axis_name="s"),
    compiler_params=pltpu.CompilerParams(needs_layout_passes=False,
                                         use_tc_tiling_on_sc=False),
    scratch_types=[
        pltpu.VMEM((_RPW,), jnp.int32),
        pltpu.VMEM((_RPW,), jnp.int32),
        pltpu.VMEM((_RPW * _NNEG,), jnp.int32),
        pltpu.VMEM((_NNEG * _RPW,), jnp.int32),
        pltpu.VMEM((_RPW, _D), jnp.float32),
        pltpu.VMEM((_RPW, _D), jnp.float32),
        pltpu.VMEM((_RING, _RPW, _D), jnp.float32),
        pltpu.VMEM((_RPW,), jnp.float32),
        pltpu.VMEM((_RPW,), jnp.float32),
        pltpu.VMEM((_RPW,), jnp.float32),
        pltpu.VMEM((_RPW,), jnp.float32),
        pltpu.VMEM((_NNEG * _RPW,), jnp.float32),
        pltpu.SemaphoreType.DMA,
        pltpu.SemaphoreType.DMA,
    ] + [pltpu.SemaphoreType.DMA] * _RING,
)(_bpr_body)


@jax.jit
def kernel(users, pos_items, neg_items, user_factors, item_factors):
  users = users.astype(jnp.int32)
  pos_items = pos_items.astype(jnp.int32)
  neg_flat = neg_items.astype(jnp.int32).reshape(-1)
  ufl, ifl = _conv(user_factors.T, item_factors.T)
  uf2 = ufl.reshape(_NROWP, _D)
  if2 = ifl.reshape(_NROWP, _D)
  acc, pqu, ppi, npi = _bpr(users, pos_items, neg_flat, uf2, if2)
  return (acc, (pqu, ppi, pqu, npi))


# speedup vs baseline: 4.0880x; 1.0043x over previous
import functools

import jax
import jax.numpy as jnp
from jax import lax
from jax.experimental import pallas as pl
from jax.experimental.pallas import tpu as pltpu
from jax.experimental.pallas import tpu_sc as plsc

_B = 16384
_D = 16
_NNEG = 20
_NW = 32
_RPW = _B // _NW
_NG = _RPW // _D
_RING = 8

_NROW = 1000000
_NBLK = 7813
_NROWP = _NBLK * 128
_CBLK = 7
_NCHK = 35
_CW = _CBLK * 128
_CWORDS = _CW * _D


def _conv_body(uft_h, ift_h, ufl_h, ifl_h, tb0_v, tb1_v, ob0_v, ob1_v,
               sem_i0, sem_i1, sem_o0, sem_o1):
  tb_refs = (tb0_v, tb1_v)
  ob_refs = (ob0_v, ob1_v)
  cid = lax.axis_index("c")
  sid = lax.axis_index("s")
  wid = sid * 2 + cid
  start = jnp.where(wid < 31, wid * 245, _NBLK - 245)
  iota = lax.iota(jnp.int32, _D)
  sem_i = (sem_i0, sem_i1)
  sem_o = (sem_o0, sem_o1)

  for src_h, dst_h in ((uft_h, ufl_h), (ift_h, ifl_h)):

    def in_copy(k, s):
      return pltpu.make_async_copy(
          src_h.at[:, pl.ds((start + k * _CBLK) * 128, _CW)],
          tb_refs[s], sem_i[s])

    def out_copy(k, s):
      return pltpu.make_async_copy(
          ob_refs[s],
          dst_h.at[pl.ds((start + k * _CBLK) * (128 * _D), _CWORDS)],
          sem_o[s])

    in_copy(0, 0).start()

    def step(k0, _):
      for s in range(2):
        k = k0 * 2 + s

        @pl.when(k < _NCHK)
        def _():
          @pl.when(k + 1 < _NCHK)
          def _():
            in_copy(k + 1, 1 - s).start()

          in_copy(k, s).wait()

          @pl.when(k >= 2)
          def _():
            out_copy(k - 2, s).wait()

          tbc = tb_refs[s]
          ob = ob_refs[s]

          @plsc.parallel_loop(0, _CBLK * 8 * 16, unroll=4)
          def _rearrange(q):
            rows = (iota + q) & (_D - 1)
            cols = iota + ((q >> 4) << 4)
            val = plsc.load_gather(tbc, [rows, cols])
            dst = ((q >> 4) << 8) + (iota << 4) + rows
            plsc.store_scatter(ob, [dst], val)

          out_copy(k, s).start()
      return 0

    lax.fori_loop(0, (_NCHK + 1) // 2, step, 0)
    out_copy(_NCHK - 2, 1).wait()
    out_copy(_NCHK - 1, 0).wait()


_conv = functools.partial(
    pl.kernel,
    out_type=[
        jax.ShapeDtypeStruct((_NROWP * _D,), jnp.float32),
        jax.ShapeDtypeStruct((_NROWP * _D,), jnp.float32),
    ],
    mesh=plsc.VectorSubcoreMesh(core_axis_name="c", subcore_axis_name="s"),
    compiler_params=pltpu.CompilerParams(needs_layout_passes=False,
                                         use_tc_tiling_on_sc=True),
    scratch_types=[
        pltpu.VMEM((_D, _CW), jnp.float32),
        pltpu.VMEM((_D, _CW), jnp.float32),
        pltpu.VMEM((_CWORDS,), jnp.float32),
        pltpu.VMEM((_CWORDS,), jnp.float32),
        pltpu.SemaphoreType.DMA,
        pltpu.SemaphoreType.DMA,
        pltpu.SemaphoreType.DMA,
        pltpu.SemaphoreType.DMA,
    ],
)(_conv_body)


def _bpr_body(users_h, pos_h, neg_h, uf_h, if_h,
              acc_h, pqu_h, ppi_h, npi_h,
              uidx_v, pidx_v, nidx2_v, nidx_v, u_v, p_v, neg_v,
              pos_v, pqu_v, ppi_v, npi_v, acc_v,
              sem_u, sem_p, *sem_n):
  cid = lax.axis_index("c")
  sid = lax.axis_index("s")
  wid = sid * 2 + cid
  base = wid * _RPW

  iota = lax.iota(jnp.int32, _D)

  pltpu.sync_copy(users_h.at[pl.ds(base, _RPW)], uidx_v)
  pltpu.sync_copy(pos_h.at[pl.ds(base, _RPW)], pidx_v)
  pltpu.sync_copy(neg_h.at[pl.ds(base * _NNEG, _RPW * _NNEG)], nidx2_v)

  du = pltpu.async_copy(uf_h.at[uidx_v], u_v, sem_u)
  dp = pltpu.async_copy(if_h.at[pidx_v], p_v, sem_p)

  @plsc.parallel_loop(0, _NNEG * _NG, unroll=4)
  def _tr_loop(g):
    n = g >> 5
    j = g & (_NG - 1)
    v = plsc.load_gather(nidx2_v, [(iota + j * _D) * _NNEG + n])
    nidx_v[pl.ds(n * _RPW + j * _D, _D)] = v

  def fire(n):
    return pltpu.async_copy(
        if_h.at[nidx_v.at[pl.ds(n * _RPW, _RPW)]],
        neg_v.at[n % _RING], sem_n[n % _RING])

  ndescs = [fire(n) for n in range(_RING)]

  du.wait()
  dp.wait()

  @plsc.parallel_loop(0, _NG, unroll=2)
  def _pos_loop(j):
    rows = iota + j * _D
    s_up = jnp.zeros((_D,), jnp.float32)
    s_uu = jnp.zeros((_D,), jnp.float32)
    s_pp = jnp.zeros((_D,), jnp.float32)
    for d in range(_D):
      cols = (iota + d) & (_D - 1)
      uv = plsc.load_gather(u_v, [rows, cols])
      pv = plsc.load_gather(p_v, [rows, cols])
      s_up = s_up + uv * pv
      s_uu = s_uu + uv * uv
      s_pp = s_pp + pv * pv
    r0 = j * _D
    pos_v[pl.ds(r0, _D)] = s_up
    pqu_v[pl.ds(r0, _D)] = s_uu
    ppi_v[pl.ds(r0, _D)] = s_pp

  for n in range(_NNEG):
    slot = n % _RING
    ndescs[n].wait()
    nslot_v = neg_v.at[slot]

    @plsc.parallel_loop(0, _NG, unroll=2)
    def _neg_loop(j):
      rows = iota + j * _D
      s0 = jnp.zeros((_D,), jnp.float32)
      s1 = jnp.zeros((_D,), jnp.float32)
      for d in range(0, _D, 2):
        cols0 = (iota + d) & (_D - 1)
        cols1 = (iota + d + 1) & (_D - 1)
        s0 = s0 + (plsc.load_gather(u_v, [rows, cols0])
                   * plsc.load_gather(nslot_v, [rows, cols0]))
        s1 = s1 + (plsc.load_gather(u_v, [rows, cols1])
                   * plsc.load_gather(nslot_v, [rows, cols1]))
      pos = pos_v[pl.ds(j * _D, _D)]
      acc_v[pl.ds(n * _RPW + j * _D, _D)] = pos - (s0 + s1)

    if n == _NNEG - 1:
      @plsc.parallel_loop(0, _NG, unroll=2)
      def _npi_loop(j):
        rows = iota + j * _D
        s_nn = jnp.zeros((_D,), jnp.float32)
        for d in range(_D):
          cols = (iota + d) & (_D - 1)
          nv = plsc.load_gather(nslot_v, [rows, cols])
          s_nn = s_nn + nv * nv
        npi_v[pl.ds(j * _D, _D)] = s_nn

    if n + _RING < _NNEG:
      ndescs.append(fire(n + _RING))

  for n in range(_NNEG):
    pltpu.sync_copy(acc_v.at[pl.ds(n * _RPW, _RPW)],
                    acc_h.at[pl.ds(n * _B + base, _RPW)])
  pltpu.sync_copy(pqu_v, pqu_h.at[pl.ds(base, _RPW)])
  pltpu.sync_copy(ppi_v, ppi_h.at[pl.ds(base, _RPW)])
  pltpu.sync_copy(npi_v, npi_h.at[pl.ds(base, _RPW)])


_bpr = functools.partial(
    pl.kernel,
    out_type=[
        jax.ShapeDtypeStruct((_NNEG * _B,), jnp.float32),
        jax.ShapeDtypeStruct((_B,), jnp.float32),
        jax.ShapeDtypeStruct((_B,), jnp.float32),
        jax.ShapeDtypeStruct((_B,), jnp.float32),
    ],
    mesh=plsc.VectorSubcoreMesh(core_axis_name="c", subcore_axis_name="s"),
    compiler_params=pltpu.CompilerParams(needs_layout_passes=False,
                                         use_tc_tiling_on_sc=False),
    scratch_types=[
        pltpu.VMEM((_RPW,), jnp.int32),
        pltpu.VMEM((_RPW,), jnp.int32),
        pltpu.VMEM((_RPW * _NNEG,), jnp.int32),
        pltpu.VMEM((_NNEG * _RPW,), jnp.int32),
        pltpu.VMEM((_RPW, _D), jnp.float32),
        pltpu.VMEM((_RPW, _D), jnp.float32),
        pltpu.VMEM((_RING, _RPW, _D), jnp.float32),
        pltpu.VMEM((_RPW,), jnp.float32),
        pltpu.VMEM((_RPW,), jnp.float32),
        pltpu.VMEM((_RPW,), jnp.float32),
        pltpu.VMEM((_RPW,), jnp.float32),
        pltpu.VMEM((_NNEG * _RPW,), jnp.float32),
        pltpu.SemaphoreType.DMA,
        pltpu.SemaphoreType.DMA,
    ] + [pltpu.SemaphoreType.DMA] * _RING,
)(_bpr_body)


@jax.jit
def kernel(users, pos_items, neg_items, user_factors, item_factors):
  users = users.astype(jnp.int32)
  pos_items = pos_items.astype(jnp.int32)
  neg_flat = neg_items.astype(jnp.int32).reshape(-1)
  ufl, ifl = _conv(user_factors.T, item_factors.T)
  uf2 = ufl.reshape(_NROWP, _D)
  if2 = ifl.reshape(_NROWP, _D)
  acc, pqu, ppi, npi = _bpr(users, pos_items, neg_flat, uf2, if2)
  return (acc, (pqu, ppi, pqu, npi))
